# Initial kernel scaffold; baseline (speedup 1.0000x reference)
import jax
import jax.numpy as jnp
from jax.experimental import pallas as pl


def kernel(input, edge, W0, a0, W1, a1, Wl, bl):
    raise NotImplementedError("write your pallas kernel here")


# speedup vs baseline: 9.9942x; 9.9942x over previous
import functools

import jax
import jax.numpy as jnp
from jax import lax
from jax.experimental import pallas as pl
from jax.experimental.pallas import tpu as pltpu
from jax.experimental.pallas import tpu_sc as plsc

N = 10000
E = 160000
IN_DIM = 128
HID = 64
OUT_DIM = 40
ALPHA = 0.2

NS = 10240
ES = 163840
CH = 128
ROWS_PER_TILE = (ES // CH) // 32
NODES_PER_TILE = NS // 16

_f32 = jnp.float32


def _sc_params():
    import dataclasses
    cp = pltpu.CompilerParams()
    fields = pltpu.CompilerParams.__dataclass_fields__
    if "needs_layout_passes" in fields:
        cp = dataclasses.replace(cp, needs_layout_passes=False)
    if "use_tc_tiling_on_sc" in fields:
        cp = dataclasses.replace(cp, use_tc_tiling_on_sc=False)
    return cp


def _edge_aggregate(g, esed_or_ex, src2, dst2, zrows, d, compute_ex):
    mesh = plsc.VectorSubcoreMesh(core_axis_name="c", subcore_axis_name="s")

    out_type = [jax.ShapeDtypeStruct((2, NS, d), _f32)]
    scratch = [
        pltpu.VMEM((ROWS_PER_TILE, CH), jnp.int32),
        pltpu.VMEM((ROWS_PER_TILE, CH), jnp.int32),
        pltpu.VMEM((ROWS_PER_TILE, CH), _f32),
        pltpu.VMEM((CH, d), _f32),
        pltpu.VMEM_SHARED((NS, d), _f32),
    ]
    if compute_ex:
        out_type += [
            jax.ShapeDtypeStruct((2, NS), _f32),
            jax.ShapeDtypeStruct((ES // CH, CH), _f32),
        ]
        scratch += [
            pltpu.VMEM((NS,), _f32),
            pltpu.VMEM((NS,), _f32),
            pltpu.VMEM((NODES_PER_TILE,), _f32),
            pltpu.VMEM_SHARED((NS,), _f32),
        ]

    @functools.partial(
        pl.kernel,
        out_type=out_type,
        mesh=mesh,
        compiler_params=_sc_params(),
        scratch_types=scratch,
    )
    def kern(*refs):
        if compute_ex:
            (g_h, es_h, ed_h, src_h, dst_h, z_h, p_h, s_h, ex_h,
             src_v, dst_v, ex_v, rows_v, p_sh, tab_s, tab_d, z_v, s_sh) = refs
        else:
            (g_h, exin_h, src_h, dst_h, z_h, p_h,
             src_v, dst_v, ex_v, rows_v, p_sh) = refs
        cid = lax.axis_index("c")
        sid = lax.axis_index("s")
        wid = cid * 16 + sid
        rbase = wid * ROWS_PER_TILE
        nbase = sid * NODES_PER_TILE

        pltpu.sync_copy(src_h.at[pl.ds(rbase, ROWS_PER_TILE)], src_v)
        pltpu.sync_copy(dst_h.at[pl.ds(rbase, ROWS_PER_TILE)], dst_v)

        pltpu.sync_copy(z_h, p_sh.at[pl.ds(nbase, NODES_PER_TILE)])

        if compute_ex:
            pltpu.sync_copy(es_h, tab_s)
            pltpu.sync_copy(ed_h, tab_d)

            @pl.loop(0, NODES_PER_TILE, step=16)
            def _(i):
                z_v[pl.ds(i, 16)] = jnp.zeros((16,), _f32)

            pltpu.sync_copy(z_v, s_sh.at[pl.ds(nbase, NODES_PER_TILE)])
        else:
            pltpu.sync_copy(exin_h.at[pl.ds(rbase, ROWS_PER_TILE)], ex_v)
        plsc.subcore_barrier()

        if compute_ex:
            @pl.loop(0, ROWS_PER_TILE)
            def _(j):
                for c in range(CH // 16):
                    sidx = src_v[j, pl.ds(c * 16, 16)]
                    didx = dst_v[j, pl.ds(c * 16, 16)]
                    e = (plsc.load_gather(tab_s, [sidx])
                         + plsc.load_gather(tab_d, [didx]))
                    v = jnp.where(e > 0.0, e, ALPHA * e)
                    ex_v[j, pl.ds(c * 16, 16)] = jnp.exp(v)
                pltpu.sync_copy(ex_v.at[j], s_sh.at[src_v.at[j]], add=True)

        @pl.loop(0, ROWS_PER_TILE)
        def _(j):
            pltpu.sync_copy(g_h.at[dst_v.at[j]], rows_v)

            @pl.loop(0, CH, step=16)
            def _(rr):
                a16 = ex_v[j, pl.ds(rr, 16)]
                for t in range(16):
                    a = a16[t]
                    for q in range(d // 16):
                        rows_v[rr + t, pl.ds(q * 16, 16)] = (
                            rows_v[rr + t, pl.ds(q * 16, 16)] * a)

            pltpu.sync_copy(rows_v, p_sh.at[src_v.at[j]], add=True)

        plsc.subcore_barrier()

        pltpu.sync_copy(p_sh.at[pl.ds(nbase, NODES_PER_TILE)],
                        p_h.at[cid, pl.ds(nbase, NODES_PER_TILE)])
        if compute_ex:
            pltpu.sync_copy(s_sh.at[pl.ds(nbase, NODES_PER_TILE)],
                            s_h.at[cid, pl.ds(nbase, NODES_PER_TILE)])
            pltpu.sync_copy(ex_v, ex_h.at[pl.ds(rbase, ROWS_PER_TILE)])

    return kern(g, *esed_or_ex, src2, dst2, zrows)


def _mm_proj(x, W, at, ab, rblk=1280):
    ns, k = x.shape
    dout = W.shape[1]

    def body(x_ref, w_ref, at_ref, ab_ref, g_ref, es_ref, ed_ref):
        g = jnp.dot(x_ref[...], w_ref[...], preferred_element_type=_f32)
        g_ref[...] = g
        es_ref[...] = jnp.dot(g, at_ref[...], preferred_element_type=_f32)
        ed_ref[...] = jnp.dot(g, ab_ref[...], preferred_element_type=_f32)

    return pl.pallas_call(
        body,
        grid=(ns // rblk,),
        in_specs=[
            pl.BlockSpec((rblk, k), lambda i: (i, 0)),
            pl.BlockSpec((k, dout), lambda i: (0, 0)),
            pl.BlockSpec((dout, 1), lambda i: (0, 0)),
            pl.BlockSpec((dout, 1), lambda i: (0, 0)),
        ],
        out_specs=[
            pl.BlockSpec((rblk, dout), lambda i: (i, 0)),
            pl.BlockSpec((rblk, 1), lambda i: (i, 0)),
            pl.BlockSpec((rblk, 1), lambda i: (i, 0)),
        ],
        out_shape=[
            jax.ShapeDtypeStruct((ns, dout), _f32),
            jax.ShapeDtypeStruct((ns, 1), _f32),
            jax.ShapeDtypeStruct((ns, 1), _f32),
        ],
    )(x, W, at, ab)


def _combine_mm(p, s, W, at, ab, rblk=1280):
    ns = p.shape[1]
    d = p.shape[2]
    dout = W.shape[1]

    def body(p_ref, s_ref, w_ref, at_ref, ab_ref, g_ref, es_ref, ed_ref):
        num = p_ref[0] + p_ref[1]
        den = s_ref[0] + s_ref[1]
        inv = jnp.where(den > 0.0, 1.0 / den, 0.0)
        h = num * inv
        h = jnp.where(h > 0.0, h, jnp.exp(h) - 1.0)
        g = jnp.dot(h, w_ref[...], preferred_element_type=_f32)
        g_ref[...] = g
        es_ref[...] = jnp.dot(g, at_ref[...], preferred_element_type=_f32)
        ed_ref[...] = jnp.dot(g, ab_ref[...], preferred_element_type=_f32)

    return pl.pallas_call(
        body,
        grid=(ns // rblk,),
        in_specs=[
            pl.BlockSpec((2, rblk, d), lambda i: (0, i, 0)),
            pl.BlockSpec((2, rblk, 1), lambda i: (0, i, 0)),
            pl.BlockSpec((d, dout), lambda i: (0, 0)),
            pl.BlockSpec((dout, 1), lambda i: (0, 0)),
            pl.BlockSpec((dout, 1), lambda i: (0, 0)),
        ],
        out_specs=[
            pl.BlockSpec((rblk, dout), lambda i: (i, 0)),
            pl.BlockSpec((rblk, 1), lambda i: (i, 0)),
            pl.BlockSpec((rblk, 1), lambda i: (i, 0)),
        ],
        out_shape=[
            jax.ShapeDtypeStruct((ns, dout), _f32),
            jax.ShapeDtypeStruct((ns, 1), _f32),
            jax.ShapeDtypeStruct((ns, 1), _f32),
        ],
    )(p, s, W, at, ab)


def _final(p, s, Wl, bl, rblk=1280):
    ns = p.shape[1]
    d = p.shape[2]
    dout = Wl.shape[1]

    def body(p_ref, s_ref, w_ref, b_ref, o_ref):
        num = p_ref[0] + p_ref[1]
        den = s_ref[0] + s_ref[1]
        inv = jnp.where(den > 0.0, 1.0 / den, 0.0)
        h = num * inv
        h = jnp.where(h > 0.0, h, jnp.exp(h) - 1.0)
        logits = jnp.dot(h, w_ref[...], preferred_element_type=_f32) + b_ref[...]
        m = jnp.max(logits, axis=1, keepdims=True)
        lse = jnp.log(jnp.sum(jnp.exp(logits - m), axis=1, keepdims=True)) + m
        o_ref[...] = logits - lse

    return pl.pallas_call(
        body,
        grid=(ns // rblk,),
        in_specs=[
            pl.BlockSpec((2, rblk, d), lambda i: (0, i, 0)),
            pl.BlockSpec((2, rblk, 1), lambda i: (0, i, 0)),
            pl.BlockSpec((d, dout), lambda i: (0, 0)),
            pl.BlockSpec((1, dout), lambda i: (0, 0)),
        ],
        out_specs=pl.BlockSpec((rblk, dout), lambda i: (i, 0)),
        out_shape=jax.ShapeDtypeStruct((ns, dout), _f32),
    )(p, s, Wl, bl)


def kernel(input, edge, W0, a0, W1, a1, Wl, bl):
    x = jnp.pad(input.astype(_f32), ((0, NS - N), (0, 0)))
    src = edge[0].astype(jnp.int32)
    dst = edge[1].astype(jnp.int32)
    pad = ES - E
    pad_idx = N + (jnp.arange(pad, dtype=jnp.int32) % (NS - N))
    src2 = jnp.concatenate([src, pad_idx]).reshape(ES // CH, CH)
    dst2 = jnp.concatenate([dst, pad_idx]).reshape(ES // CH, CH)

    zrows = jnp.zeros((NODES_PER_TILE, HID), _f32)

    g0, es0, ed0 = _mm_proj(x, W0, a0[: 2 * HID], a0[2 * HID:])
    pA, s0, ex0 = _edge_aggregate(g0[:, :HID], (es0.reshape(NS), ed0.reshape(NS)),
                                  src2, dst2, zrows, HID, True)
    (pB,) = _edge_aggregate(g0[:, HID:], (ex0,), src2, dst2, zrows, HID, False)
    p0 = jnp.concatenate([pA, pB], axis=2)
    g1, es1, ed1 = _combine_mm(p0, s0.reshape(2, NS, 1), W1, a1[:HID], a1[HID:])
    p1, s1, _ = _edge_aggregate(g1, (es1.reshape(NS), ed1.reshape(NS)),
                                src2, dst2, zrows, HID, True)
    out = _final(p1, s1.reshape(2, NS, 1), Wl, bl.reshape(1, OUT_DIM))
    return out[:N]


# speedup vs baseline: 20.5600x; 2.0572x over previous
import functools

import jax
import jax.numpy as jnp
from jax import lax
from jax.experimental import pallas as pl
from jax.experimental.pallas import tpu as pltpu
from jax.experimental.pallas import tpu_sc as plsc

N = 10000
E = 160000
IN_DIM = 128
HID = 64
OUT_DIM = 40
ALPHA = 0.2

NS = 10240
ES = 163840
CH = 128
ROWS_PER_TILE = (ES // CH) // 32
NODES_PER_TILE = NS // 16

_f32 = jnp.float32


def _sc_params():
    import dataclasses
    cp = pltpu.CompilerParams()
    fields = pltpu.CompilerParams.__dataclass_fields__
    if "needs_layout_passes" in fields:
        cp = dataclasses.replace(cp, needs_layout_passes=False)
    if "use_tc_tiling_on_sc" in fields:
        cp = dataclasses.replace(cp, use_tc_tiling_on_sc=False)
    return cp


def _edge_aggregate2(g2, es, ed, src2, dst2, zrows, dh):
    mesh = plsc.VectorSubcoreMesh(core_axis_name="c", subcore_axis_name="s")
    R = (ES // CH) // 16
    NBUF = 2

    @functools.partial(
        pl.kernel,
        out_type=[
            jax.ShapeDtypeStruct((2, NS, dh), _f32),
            jax.ShapeDtypeStruct((2, NS), _f32),
        ],
        mesh=mesh,
        compiler_params=_sc_params(),
        scratch_types=[
            pltpu.VMEM((R, CH), jnp.int32),
            pltpu.VMEM((R, CH), jnp.int32),
            pltpu.VMEM((R, CH), _f32),
            pltpu.VMEM((NS,), _f32),
            pltpu.VMEM((NS,), _f32),
            [pltpu.VMEM((CH, dh), _f32) for _ in range(NBUF)],
            [pltpu.VMEM((CH, dh), _f32) for _ in range(NBUF)],
            pltpu.VMEM((NODES_PER_TILE,), _f32),
            pltpu.VMEM_SHARED((NS, dh), _f32),
            pltpu.VMEM_SHARED((NS,), _f32),
            [pltpu.SemaphoreType.DMA for _ in range(NBUF)],
            [pltpu.SemaphoreType.DMA for _ in range(NBUF)],
            pltpu.SemaphoreType.DMA,
        ],
    )
    def kern(g_h, es_h, ed_h, src_h, dst_h, z_h, p_h, s_h,
             src_v, dst_v, ex_v, tab_s, tab_d, gbuf, sbuf, z_v, p_sh, s_sh,
             gsem, ssem, xsem):
        cid = lax.axis_index("c")
        sid = lax.axis_index("s")
        rbase = sid * R
        nbase = sid * NODES_PER_TILE

        pltpu.sync_copy(src_h.at[pl.ds(rbase, R)], src_v)
        pltpu.sync_copy(dst_h.at[pl.ds(rbase, R)], dst_v)
        pltpu.sync_copy(es_h, tab_s)
        pltpu.sync_copy(ed_h, tab_d)

        off = (cid * NS).astype(jnp.int32)

        @pl.loop(0, R)
        def _(j):
            for c in range(CH // 16):
                dst_v[j, pl.ds(c * 16, 16)] = dst_v[j, pl.ds(c * 16, 16)] + off

        pltpu.sync_copy(z_h, p_sh.at[pl.ds(nbase, NODES_PER_TILE)])

        @pl.loop(0, NODES_PER_TILE, step=16)
        def _(i):
            z_v[pl.ds(i, 16)] = jnp.zeros((16,), _f32)

        pltpu.sync_copy(z_v, s_sh.at[pl.ds(nbase, NODES_PER_TILE)])

        for b in range(NBUF):
            pltpu.async_copy(g_h.at[dst_v.at[b]], gbuf[b], gsem[b])

        plsc.subcore_barrier()

        @pl.loop(0, R, step=NBUF)
        def _(j0):
            for b in range(NBUF):
                j = j0 + b
                for c in range(CH // 16):
                    sidx = src_v[j, pl.ds(c * 16, 16)]
                    didx = dst_v[j, pl.ds(c * 16, 16)] - off
                    e = (plsc.load_gather(tab_s, [sidx])
                         + plsc.load_gather(tab_d, [didx]))
                    v = jnp.where(e > 0.0, e, ALPHA * e)
                    ex_v[j, pl.ds(c * 16, 16)] = jnp.exp(v)
                pltpu.async_copy(ex_v.at[j], s_sh.at[src_v.at[j]], xsem,
                                 add=True)

                pltpu.make_async_copy(g_h.at[dst_v.at[j]], gbuf[b],
                                      gsem[b]).wait()
                @pl.when(j0 >= NBUF)
                def _():
                    pltpu.make_async_copy(
                        sbuf[b], p_sh.at[src_v.at[j - NBUF]], ssem[b]).wait()

                @pl.loop(0, CH, step=16)
                def _(rr):
                    a16 = ex_v[j, pl.ds(rr, 16)]
                    for t in range(16):
                        a = a16[t]
                        for q in range(dh // 16):
                            sbuf[b][rr + t, pl.ds(q * 16, 16)] = (
                                gbuf[b][rr + t, pl.ds(q * 16, 16)] * a)

                pltpu.async_copy(sbuf[b], p_sh.at[src_v.at[j]], ssem[b],
                                 add=True)

                @pl.when(j + NBUF < R)
                def _():
                    pltpu.async_copy(g_h.at[dst_v.at[j + NBUF]], gbuf[b],
                                     gsem[b])

        for b in range(NBUF):
            pltpu.make_async_copy(sbuf[b], p_sh.at[src_v.at[R - NBUF + b]],
                                  ssem[b]).wait()

        @pl.loop(0, R)
        def _(j):
            pltpu.make_async_copy(ex_v.at[j], s_sh.at[src_v.at[j]],
                                  xsem).wait()

        plsc.subcore_barrier()

        pltpu.sync_copy(p_sh.at[pl.ds(nbase, NODES_PER_TILE)],
                        p_h.at[cid, pl.ds(nbase, NODES_PER_TILE)])
        pltpu.sync_copy(s_sh.at[pl.ds(nbase, NODES_PER_TILE)],
                        s_h.at[cid, pl.ds(nbase, NODES_PER_TILE)])

    return kern(g2, es, ed, src2, dst2, zrows)


def _mm_proj(x, W, at, ab, rblk=1280):
    ns, k = x.shape
    dout = W.shape[1]

    def body(x_ref, w_ref, at_ref, ab_ref, g_ref, es_ref, ed_ref):
        g = jnp.dot(x_ref[...], w_ref[...], preferred_element_type=_f32)
        g_ref[...] = g
        es_ref[...] = jnp.dot(g, at_ref[...], preferred_element_type=_f32)
        ed_ref[...] = jnp.dot(g, ab_ref[...], preferred_element_type=_f32)

    return pl.pallas_call(
        body,
        grid=(ns // rblk,),
        in_specs=[
            pl.BlockSpec((rblk, k), lambda i: (i, 0)),
            pl.BlockSpec((k, dout), lambda i: (0, 0)),
            pl.BlockSpec((dout, 1), lambda i: (0, 0)),
            pl.BlockSpec((dout, 1), lambda i: (0, 0)),
        ],
        out_specs=[
            pl.BlockSpec((rblk, dout), lambda i: (i, 0)),
            pl.BlockSpec((rblk, 1), lambda i: (i, 0)),
            pl.BlockSpec((rblk, 1), lambda i: (i, 0)),
        ],
        out_shape=[
            jax.ShapeDtypeStruct((ns, dout), _f32),
            jax.ShapeDtypeStruct((ns, 1), _f32),
            jax.ShapeDtypeStruct((ns, 1), _f32),
        ],
    )(x, W, at, ab)


def _combine_mm(p, s, W, at, ab, rblk=1280):
    ns = p.shape[1]
    dh = p.shape[2]
    dout = W.shape[1]

    def body(p_ref, s_ref, w_ref, at_ref, ab_ref, g_ref, es_ref, ed_ref):
        den = s_ref[0]
        inv = jnp.where(den > 0.0, 1.0 / den, 0.0)
        ha = p_ref[0] * inv
        ha = jnp.where(ha > 0.0, ha, jnp.exp(ha) - 1.0)
        hb = p_ref[1] * inv
        hb = jnp.where(hb > 0.0, hb, jnp.exp(hb) - 1.0)
        w = w_ref[...]
        g = (jnp.dot(ha, w[:dh], preferred_element_type=_f32)
             + jnp.dot(hb, w[dh:], preferred_element_type=_f32))
        g_ref[...] = g
        es_ref[...] = jnp.dot(g, at_ref[...], preferred_element_type=_f32)
        ed_ref[...] = jnp.dot(g, ab_ref[...], preferred_element_type=_f32)

    return pl.pallas_call(
        body,
        grid=(ns // rblk,),
        in_specs=[
            pl.BlockSpec((2, rblk, dh), lambda i: (0, i, 0)),
            pl.BlockSpec((2, rblk, 1), lambda i: (0, i, 0)),
            pl.BlockSpec((2 * dh, dout), lambda i: (0, 0)),
            pl.BlockSpec((dout, 1), lambda i: (0, 0)),
            pl.BlockSpec((dout, 1), lambda i: (0, 0)),
        ],
        out_specs=[
            pl.BlockSpec((rblk, dout), lambda i: (i, 0)),
            pl.BlockSpec((rblk, 1), lambda i: (i, 0)),
            pl.BlockSpec((rblk, 1), lambda i: (i, 0)),
        ],
        out_shape=[
            jax.ShapeDtypeStruct((ns, dout), _f32),
            jax.ShapeDtypeStruct((ns, 1), _f32),
            jax.ShapeDtypeStruct((ns, 1), _f32),
        ],
    )(p, s, W, at, ab)


def _final(p, s, Wl, bl, rblk=1280):
    ns = p.shape[1]
    dh = p.shape[2]
    dout = Wl.shape[1]

    def body(p_ref, s_ref, w_ref, b_ref, o_ref):
        den = s_ref[0]
        inv = jnp.where(den > 0.0, 1.0 / den, 0.0)
        ha = p_ref[0] * inv
        ha = jnp.where(ha > 0.0, ha, jnp.exp(ha) - 1.0)
        hb = p_ref[1] * inv
        hb = jnp.where(hb > 0.0, hb, jnp.exp(hb) - 1.0)
        w = w_ref[...]
        logits = (jnp.dot(ha, w[:dh], preferred_element_type=_f32)
                  + jnp.dot(hb, w[dh:], preferred_element_type=_f32)
                  + b_ref[...])
        m = jnp.max(logits, axis=1, keepdims=True)
        lse = jnp.log(jnp.sum(jnp.exp(logits - m), axis=1, keepdims=True)) + m
        o_ref[...] = logits - lse

    return pl.pallas_call(
        body,
        grid=(ns // rblk,),
        in_specs=[
            pl.BlockSpec((2, rblk, dh), lambda i: (0, i, 0)),
            pl.BlockSpec((2, rblk, 1), lambda i: (0, i, 0)),
            pl.BlockSpec((2 * dh, dout), lambda i: (0, 0)),
            pl.BlockSpec((1, dout), lambda i: (0, 0)),
        ],
        out_specs=pl.BlockSpec((rblk, dout), lambda i: (i, 0)),
        out_shape=jax.ShapeDtypeStruct((ns, dout), _f32),
    )(p, s, Wl, bl)


def kernel(input, edge, W0, a0, W1, a1, Wl, bl):
    x = jnp.pad(input.astype(_f32), ((0, NS - N), (0, 0)))
    src = edge[0].astype(jnp.int32)
    dst = edge[1].astype(jnp.int32)
    pad = ES - E
    pad_idx = N + (jnp.arange(pad, dtype=jnp.int32) % (NS - N))
    src2 = jnp.concatenate([src, pad_idx]).reshape(ES // CH, CH)
    dst2 = jnp.concatenate([dst, pad_idx]).reshape(ES // CH, CH)

    g0, es0, ed0 = _mm_proj(x, W0, a0[: 2 * HID], a0[2 * HID:])
    g0s = jnp.concatenate([g0[:, :HID], g0[:, HID:]], axis=0)
    p0, s0 = _edge_aggregate2(g0s, es0.reshape(NS), ed0.reshape(NS),
                              src2, dst2, jnp.zeros((NODES_PER_TILE, HID), _f32),
                              HID)
    g1, es1, ed1 = _combine_mm(p0, s0.reshape(2, NS, 1), W1, a1[:HID], a1[HID:])
    g1s = jnp.concatenate([g1[:, : HID // 2], g1[:, HID // 2:]], axis=0)
    p1, s1 = _edge_aggregate2(g1s, es1.reshape(NS), ed1.reshape(NS),
                              src2, dst2,
                              jnp.zeros((NODES_PER_TILE, HID // 2), _f32),
                              HID // 2)
    out = _final(p1, s1.reshape(2, NS, 1), Wl, bl.reshape(1, OUT_DIM))
    return out[:N]


# speedup vs baseline: 21.2152x; 1.0319x over previous
import functools

import jax
import jax.numpy as jnp
from jax import lax
from jax.experimental import pallas as pl
from jax.experimental.pallas import tpu as pltpu
from jax.experimental.pallas import tpu_sc as plsc

N = 10000
E = 160000
IN_DIM = 128
HID = 64
OUT_DIM = 40
ALPHA = 0.2

NS = 10240
ES = 163840
CH = 128
ROWS_PER_TILE = (ES // CH) // 32
NODES_PER_TILE = NS // 16

_f32 = jnp.float32


def _sc_params():
    import dataclasses
    cp = pltpu.CompilerParams()
    fields = pltpu.CompilerParams.__dataclass_fields__
    if "needs_layout_passes" in fields:
        cp = dataclasses.replace(cp, needs_layout_passes=False)
    if "use_tc_tiling_on_sc" in fields:
        cp = dataclasses.replace(cp, use_tc_tiling_on_sc=False)
    if "disable_bounds_checks" in fields:
        cp = dataclasses.replace(cp, disable_bounds_checks=True)
    return cp


def _edge_aggregate2(g2, es, ed, src2, dst2, zrows, dh):
    mesh = plsc.VectorSubcoreMesh(core_axis_name="c", subcore_axis_name="s")
    R = (ES // CH) // 16
    NBUF = 2

    @functools.partial(
        pl.kernel,
        out_type=[
            jax.ShapeDtypeStruct((2, NS, dh), _f32),
            jax.ShapeDtypeStruct((2, NS), _f32),
        ],
        mesh=mesh,
        compiler_params=_sc_params(),
        scratch_types=[
            pltpu.VMEM((R, CH), jnp.int32),
            pltpu.VMEM((R, CH), jnp.int32),
            pltpu.VMEM((R, CH), _f32),
            pltpu.VMEM((NS,), _f32),
            pltpu.VMEM((NS,), _f32),
            [pltpu.VMEM((CH, dh), _f32) for _ in range(NBUF)],
            [pltpu.VMEM((CH, dh), _f32) for _ in range(NBUF)],
            pltpu.VMEM((NODES_PER_TILE,), _f32),
            pltpu.VMEM_SHARED((NS, dh), _f32),
            pltpu.VMEM_SHARED((NS,), _f32),
            [pltpu.SemaphoreType.DMA for _ in range(NBUF)],
            [pltpu.SemaphoreType.DMA for _ in range(NBUF)],
            pltpu.SemaphoreType.DMA,
            pltpu.SemaphoreType.DMA,
        ],
    )
    def kern(g_h, es_h, ed_h, src_h, dst_h, z_h, p_h, s_h,
             src_v, dst_v, ex_v, tab_s, tab_d, gbuf, sbuf, z_v, p_sh, s_sh,
             gsem, ssem, xsem, lsem):
        cid = lax.axis_index("c")
        sid = lax.axis_index("s")
        rbase = sid * R
        nbase = sid * NODES_PER_TILE

        ld = [
            pltpu.async_copy(src_h.at[pl.ds(rbase, R)], src_v, lsem),
            pltpu.async_copy(dst_h.at[pl.ds(rbase, R)], dst_v, lsem),
            pltpu.async_copy(es_h, tab_s, lsem),
            pltpu.async_copy(ed_h, tab_d, lsem),
        ]

        pltpu.sync_copy(z_h, p_sh.at[pl.ds(nbase, NODES_PER_TILE)])

        @pl.loop(0, NODES_PER_TILE, step=16)
        def _(i):
            z_v[pl.ds(i, 16)] = jnp.zeros((16,), _f32)

        pltpu.sync_copy(z_v, s_sh.at[pl.ds(nbase, NODES_PER_TILE)])

        for c in ld:
            c.wait()

        off = (cid * NS).astype(jnp.int32)

        @pl.loop(0, R)
        def _(j):
            for c in range(CH // 16):
                dst_v[j, pl.ds(c * 16, 16)] = dst_v[j, pl.ds(c * 16, 16)] + off

        for b in range(NBUF):
            pltpu.async_copy(g_h.at[dst_v.at[b]], gbuf[b], gsem[b])

        plsc.subcore_barrier()

        @pl.loop(0, R, step=NBUF)
        def _(j0):
            for b in range(NBUF):
                j = j0 + b
                for c in range(CH // 16):
                    sidx = src_v[j, pl.ds(c * 16, 16)]
                    didx = dst_v[j, pl.ds(c * 16, 16)] - off
                    e = (plsc.load_gather(tab_s, [sidx])
                         + plsc.load_gather(tab_d, [didx]))
                    v = jnp.where(e > 0.0, e, ALPHA * e)
                    ex_v[j, pl.ds(c * 16, 16)] = jnp.exp(v)
                pltpu.async_copy(ex_v.at[j], s_sh.at[src_v.at[j]], xsem,
                                 add=True)

                pltpu.make_async_copy(g_h.at[dst_v.at[j]], gbuf[b],
                                      gsem[b]).wait()
                @pl.when(j0 >= NBUF)
                def _():
                    pltpu.make_async_copy(
                        sbuf[b], p_sh.at[src_v.at[j - NBUF]], ssem[b]).wait()

                @pl.loop(0, CH, step=16)
                def _(rr):
                    a16 = ex_v[j, pl.ds(rr, 16)]
                    for t in range(16):
                        a = a16[t]
                        for q in range(dh // 16):
                            sbuf[b][rr + t, pl.ds(q * 16, 16)] = (
                                gbuf[b][rr + t, pl.ds(q * 16, 16)] * a)

                pltpu.async_copy(sbuf[b], p_sh.at[src_v.at[j]], ssem[b],
                                 add=True)

                @pl.when(j + NBUF < R)
                def _():
                    pltpu.async_copy(g_h.at[dst_v.at[j + NBUF]], gbuf[b],
                                     gsem[b])

        for b in range(NBUF):
            pltpu.make_async_copy(sbuf[b], p_sh.at[src_v.at[R - NBUF + b]],
                                  ssem[b]).wait()

        @pl.loop(0, R)
        def _(j):
            pltpu.make_async_copy(ex_v.at[j], s_sh.at[src_v.at[j]],
                                  xsem).wait()

        plsc.subcore_barrier()

        pltpu.sync_copy(p_sh.at[pl.ds(nbase, NODES_PER_TILE)],
                        p_h.at[cid, pl.ds(nbase, NODES_PER_TILE)])
        pltpu.sync_copy(s_sh.at[pl.ds(nbase, NODES_PER_TILE)],
                        s_h.at[cid, pl.ds(nbase, NODES_PER_TILE)])

    return kern(g2, es, ed, src2, dst2, zrows)


def _mm_proj(x, W, at, ab, rblk=1280):
    ns, k = x.shape
    dout = W.shape[1]
    dh = dout // 2
    nb = ns // rblk
    Ws = jnp.stack([W[:, :dh], W[:, dh:]])

    def body(x_ref, w_ref, at_ref, ab_ref, g_ref, es_ref, ed_ref):
        j = pl.program_id(1)
        g = jnp.dot(x_ref[...], w_ref[0], preferred_element_type=_f32)
        g_ref[...] = g
        es = jnp.dot(g, at_ref[...], preferred_element_type=_f32)
        ed = jnp.dot(g, ab_ref[...], preferred_element_type=_f32)

        @pl.when(j == 0)
        def _():
            es_ref[...] = es
            ed_ref[...] = ed

        @pl.when(j == 1)
        def _():
            es_ref[...] += es
            ed_ref[...] += ed

    return pl.pallas_call(
        body,
        grid=(nb, 2),
        in_specs=[
            pl.BlockSpec((rblk, k), lambda i, j: (i, 0)),
            pl.BlockSpec((1, k, dh), lambda i, j: (j, 0, 0)),
            pl.BlockSpec((dh, 1), lambda i, j: (j, 0)),
            pl.BlockSpec((dh, 1), lambda i, j: (j, 0)),
        ],
        out_specs=[
            pl.BlockSpec((rblk, dh), lambda i, j: (j * nb + i, 0)),
            pl.BlockSpec((rblk, 1), lambda i, j: (i, 0)),
            pl.BlockSpec((rblk, 1), lambda i, j: (i, 0)),
        ],
        out_shape=[
            jax.ShapeDtypeStruct((2 * ns, dh), _f32),
            jax.ShapeDtypeStruct((ns, 1), _f32),
            jax.ShapeDtypeStruct((ns, 1), _f32),
        ],
    )(x, Ws, at, ab)


def _combine_mm(p, s, W, at, ab, rblk=1280):
    ns = p.shape[1]
    dh = p.shape[2]
    dout = W.shape[1]

    dho = dout // 2
    nb = ns // rblk
    Ws = jnp.stack([W[:, :dho], W[:, dho:]])

    def body(p_ref, s_ref, w_ref, at_ref, ab_ref, g_ref, es_ref, ed_ref):
        j = pl.program_id(1)
        den = s_ref[0]
        inv = jnp.where(den > 0.0, 1.0 / den, 0.0)
        ha = p_ref[0] * inv
        ha = jnp.where(ha > 0.0, ha, jnp.exp(ha) - 1.0)
        hb = p_ref[1] * inv
        hb = jnp.where(hb > 0.0, hb, jnp.exp(hb) - 1.0)
        w = w_ref[0]
        g = (jnp.dot(ha, w[:dh], preferred_element_type=_f32)
             + jnp.dot(hb, w[dh:], preferred_element_type=_f32))
        g_ref[...] = g
        es = jnp.dot(g, at_ref[...], preferred_element_type=_f32)
        ed = jnp.dot(g, ab_ref[...], preferred_element_type=_f32)

        @pl.when(j == 0)
        def _():
            es_ref[...] = es
            ed_ref[...] = ed

        @pl.when(j == 1)
        def _():
            es_ref[...] += es
            ed_ref[...] += ed

    return pl.pallas_call(
        body,
        grid=(nb, 2),
        in_specs=[
            pl.BlockSpec((2, rblk, dh), lambda i, j: (0, i, 0)),
            pl.BlockSpec((2, rblk, 1), lambda i, j: (0, i, 0)),
            pl.BlockSpec((1, 2 * dh, dho), lambda i, j: (j, 0, 0)),
            pl.BlockSpec((dho, 1), lambda i, j: (j, 0)),
            pl.BlockSpec((dho, 1), lambda i, j: (j, 0)),
        ],
        out_specs=[
            pl.BlockSpec((rblk, dho), lambda i, j: (j * nb + i, 0)),
            pl.BlockSpec((rblk, 1), lambda i, j: (i, 0)),
            pl.BlockSpec((rblk, 1), lambda i, j: (i, 0)),
        ],
        out_shape=[
            jax.ShapeDtypeStruct((2 * ns, dho), _f32),
            jax.ShapeDtypeStruct((ns, 1), _f32),
            jax.ShapeDtypeStruct((ns, 1), _f32),
        ],
    )(p, s, Ws, at, ab)


def _final(p, s, Wl, bl, rblk=1280):
    ns = p.shape[1]
    dh = p.shape[2]
    dout = Wl.shape[1]

    def body(p_ref, s_ref, w_ref, b_ref, o_ref):
        den = s_ref[0]
        inv = jnp.where(den > 0.0, 1.0 / den, 0.0)
        ha = p_ref[0] * inv
        ha = jnp.where(ha > 0.0, ha, jnp.exp(ha) - 1.0)
        hb = p_ref[1] * inv
        hb = jnp.where(hb > 0.0, hb, jnp.exp(hb) - 1.0)
        w = w_ref[...]
        logits = (jnp.dot(ha, w[:dh], preferred_element_type=_f32)
                  + jnp.dot(hb, w[dh:], preferred_element_type=_f32)
                  + b_ref[...])
        m = jnp.max(logits, axis=1, keepdims=True)
        lse = jnp.log(jnp.sum(jnp.exp(logits - m), axis=1, keepdims=True)) + m
        o_ref[...] = logits - lse

    return pl.pallas_call(
        body,
        grid=(ns // rblk,),
        in_specs=[
            pl.BlockSpec((2, rblk, dh), lambda i: (0, i, 0)),
            pl.BlockSpec((2, rblk, 1), lambda i: (0, i, 0)),
            pl.BlockSpec((2 * dh, dout), lambda i: (0, 0)),
            pl.BlockSpec((1, dout), lambda i: (0, 0)),
        ],
        out_specs=pl.BlockSpec((rblk, dout), lambda i: (i, 0)),
        out_shape=jax.ShapeDtypeStruct((ns, dout), _f32),
    )(p, s, Wl, bl)


def kernel(input, edge, W0, a0, W1, a1, Wl, bl):
    x = jnp.pad(input.astype(_f32), ((0, NS - N), (0, 0)))
    src = edge[0].astype(jnp.int32)
    dst = edge[1].astype(jnp.int32)
    pad = ES - E
    pad_idx = N + (jnp.arange(pad, dtype=jnp.int32) % (NS - N))
    src2 = jnp.concatenate([src, pad_idx]).reshape(ES // CH, CH)
    dst2 = jnp.concatenate([dst, pad_idx]).reshape(ES // CH, CH)

    g0s, es0, ed0 = _mm_proj(x, W0, a0[: 2 * HID], a0[2 * HID:])
    p0, s0 = _edge_aggregate2(g0s, es0.reshape(NS), ed0.reshape(NS),
                              src2, dst2, jnp.zeros((NODES_PER_TILE, HID), _f32),
                              HID)
    g1s, es1, ed1 = _combine_mm(p0, s0.reshape(2, NS, 1), W1, a1[:HID], a1[HID:])
    p1, s1 = _edge_aggregate2(g1s, es1.reshape(NS), ed1.reshape(NS),
                              src2, dst2,
                              jnp.zeros((NODES_PER_TILE, HID // 2), _f32),
                              HID // 2)
    out = _final(p1, s1.reshape(2, NS, 1), Wl, bl.reshape(1, OUT_DIM))
    return out[:N]


# speedup vs baseline: 29.1031x; 1.3718x over previous
import functools

import jax
import jax.numpy as jnp
from jax import lax
from jax.experimental import pallas as pl
from jax.experimental.pallas import tpu as pltpu
from jax.experimental.pallas import tpu_sc as plsc

N = 10000
E = 160000
IN_DIM = 128
HID = 64
OUT_DIM = 40
ALPHA = 0.2

NS = 10240
ES = 163840
CH = 128
NODES_PER_TILE = NS // 16

_f32 = jnp.float32
_DN_T = (((0,), (1,)), ((), ()))


def _sc_params():
    import dataclasses
    cp = pltpu.CompilerParams()
    fields = pltpu.CompilerParams.__dataclass_fields__
    if "needs_layout_passes" in fields:
        cp = dataclasses.replace(cp, needs_layout_passes=False)
    if "use_tc_tiling_on_sc" in fields:
        cp = dataclasses.replace(cp, use_tc_tiling_on_sc=False)
    if "disable_bounds_checks" in fields:
        cp = dataclasses.replace(cp, disable_bounds_checks=True)
    return cp


def _edge_aggregate2(g2, es, ed, e3, zrows, dh):
    mesh = plsc.VectorSubcoreMesh(core_axis_name="c", subcore_axis_name="s")
    R = (ES // CH) // 16
    NBUF = 2

    @functools.partial(
        pl.kernel,
        out_type=jax.ShapeDtypeStruct((2, NS, dh), _f32),
        mesh=mesh,
        compiler_params=_sc_params(),
        scratch_types=[
            pltpu.VMEM((R, CH), jnp.int32),
            pltpu.VMEM((R, CH), jnp.int32),
            pltpu.VMEM((R, CH), _f32),
            pltpu.VMEM((NS,), _f32),
            pltpu.VMEM((NS,), _f32),
            [pltpu.VMEM((CH, dh), _f32) for _ in range(NBUF)],
            [pltpu.VMEM((CH, dh), _f32) for _ in range(NBUF)],
            pltpu.VMEM((NODES_PER_TILE,), _f32),
            pltpu.VMEM_SHARED((NS, dh), _f32),
            pltpu.VMEM_SHARED((NS,), _f32),
            [pltpu.SemaphoreType.DMA for _ in range(NBUF)],
            [pltpu.SemaphoreType.DMA for _ in range(NBUF)],
            pltpu.SemaphoreType.DMA,
            pltpu.SemaphoreType.DMA,
        ],
    )
    def kern(g_h, es_h, ed_h, e3_h, z_h, p_h,
             src_v, dst_v, ex_v, tab_s, tab_d, gbuf, sbuf, z_v, p_sh, s_sh,
             gsem, ssem, xsem, lsem):
        cid = lax.axis_index("c")
        sid = lax.axis_index("s")
        rbase = sid * R
        nbase = sid * NODES_PER_TILE

        ld = [
            pltpu.async_copy(e3_h.at[0, pl.ds(rbase, R)], src_v, lsem),
            pltpu.async_copy(e3_h.at[1, pl.ds(rbase, R)], dst_v, lsem),
            pltpu.async_copy(es_h.at[0], tab_s, lsem),
            pltpu.async_copy(ed_h.at[0], tab_d, lsem),
        ]

        pltpu.sync_copy(z_h, p_sh.at[pl.ds(nbase, NODES_PER_TILE)])

        @pl.loop(0, NODES_PER_TILE, step=16)
        def _(i):
            z_v[pl.ds(i, 16)] = jnp.zeros((16,), _f32)

        pltpu.sync_copy(z_v, s_sh.at[pl.ds(nbase, NODES_PER_TILE)])

        for c in ld:
            c.wait()

        @pl.loop(0, R)
        def _(j):
            for c in range(CH // 16):
                dst_v[j, pl.ds(c * 16, 16)] = (
                    dst_v[j, pl.ds(c * 16, 16)] * 2 + cid)

        for b in range(NBUF):
            pltpu.async_copy(g_h.at[dst_v.at[b]], gbuf[b], gsem[b])

        plsc.subcore_barrier()

        @pl.loop(0, R, step=NBUF)
        def _(j0):
            for b in range(NBUF):
                j = j0 + b
                for c in range(CH // 16):
                    sidx = src_v[j, pl.ds(c * 16, 16)]
                    didx = lax.shift_right_logical(
                        dst_v[j, pl.ds(c * 16, 16)] - cid, 1)
                    e = (plsc.load_gather(tab_s, [sidx])
                         + plsc.load_gather(tab_d, [didx]))
                    v = jnp.where(e > 0.0, e, ALPHA * e)
                    ex_v[j, pl.ds(c * 16, 16)] = jnp.exp(v)
                pltpu.async_copy(ex_v.at[j], s_sh.at[src_v.at[j]], xsem,
                                 add=True)

                pltpu.make_async_copy(g_h.at[dst_v.at[j]], gbuf[b],
                                      gsem[b]).wait()
                @pl.when(j0 >= NBUF)
                def _():
                    pltpu.make_async_copy(
                        sbuf[b], p_sh.at[src_v.at[j - NBUF]], ssem[b]).wait()

                @pl.loop(0, CH, step=16)
                def _(rr):
                    a16 = ex_v[j, pl.ds(rr, 16)]
                    for t in range(16):
                        a = a16[t]
                        for q in range(dh // 16):
                            sbuf[b][rr + t, pl.ds(q * 16, 16)] = (
                                gbuf[b][rr + t, pl.ds(q * 16, 16)] * a)

                pltpu.async_copy(sbuf[b], p_sh.at[src_v.at[j]], ssem[b],
                                 add=True)

                @pl.when(j + NBUF < R)
                def _():
                    pltpu.async_copy(g_h.at[dst_v.at[j + NBUF]], gbuf[b],
                                     gsem[b])

        for b in range(NBUF):
            pltpu.make_async_copy(sbuf[b], p_sh.at[src_v.at[R - NBUF + b]],
                                  ssem[b]).wait()

        @pl.loop(0, R)
        def _(j):
            pltpu.make_async_copy(ex_v.at[j], s_sh.at[src_v.at[j]],
                                  xsem).wait()

        plsc.subcore_barrier()

        pltpu.sync_copy(s_sh.at[pl.ds(nbase, NODES_PER_TILE)], z_v)

        @pl.loop(0, NODES_PER_TILE, step=16)
        def _(i):
            s16 = z_v[pl.ds(i, 16)]
            z_v[pl.ds(i, 16)] = jnp.where(s16 > 0.0, 1.0 / s16, 0.0)

        @pl.loop(0, NODES_PER_TILE, step=CH)
        def _(i):
            pltpu.sync_copy(p_sh.at[pl.ds(nbase + i, CH)], gbuf[0])

            @pl.loop(0, CH, step=16)
            def _(rr):
                a16 = z_v[pl.ds(i + rr, 16)]
                for t in range(16):
                    a = a16[t]
                    for q in range(dh // 16):
                        sbuf[0][rr + t, pl.ds(q * 16, 16)] = (
                            gbuf[0][rr + t, pl.ds(q * 16, 16)] * a)

            pltpu.sync_copy(sbuf[0], p_h.at[cid, pl.ds(nbase + i, CH)])

    return kern(g2, es, ed, e3, zrows)


def _mm_proj(x, W, at, ab, rblk=1280):
    ns, k = x.shape
    dout = W.shape[1]

    def body(x_ref, w_ref, at_ref, ab_ref, g_ref, es_ref, ed_ref):
        g = jnp.dot(x_ref[...], w_ref[...], preferred_element_type=_f32)
        g_ref[...] = g
        es_ref[...] = lax.dot_general(at_ref[...], g, _DN_T,
                                      preferred_element_type=_f32)
        ed_ref[...] = lax.dot_general(ab_ref[...], g, _DN_T,
                                      preferred_element_type=_f32)

    return pl.pallas_call(
        body,
        grid=(ns // rblk,),
        in_specs=[
            pl.BlockSpec((rblk, k), lambda i: (i, 0)),
            pl.BlockSpec((k, dout), lambda i: (0, 0)),
            pl.BlockSpec((k, 1), lambda i: (0, 0)),
            pl.BlockSpec((k, 1), lambda i: (0, 0)),
        ],
        out_specs=[
            pl.BlockSpec((rblk, dout), lambda i: (i, 0)),
            pl.BlockSpec((1, rblk), lambda i: (0, i)),
            pl.BlockSpec((1, rblk), lambda i: (0, i)),
        ],
        out_shape=[
            jax.ShapeDtypeStruct((ns, dout), _f32),
            jax.ShapeDtypeStruct((1, ns), _f32),
            jax.ShapeDtypeStruct((1, ns), _f32),
        ],
    )(x, W, at, ab)


def _combine_mm(p, W, at, ab, rblk=1280):
    ns = p.shape[1]
    dh = p.shape[2]
    dout = W.shape[1]

    def body(p_ref, w_ref, at_ref, ab_ref, g_ref, es_ref, ed_ref):
        ha = p_ref[0]
        ha = jnp.where(ha > 0.0, ha, jnp.exp(ha) - 1.0)
        hb = p_ref[1]
        hb = jnp.where(hb > 0.0, hb, jnp.exp(hb) - 1.0)
        w = w_ref[...]
        g = (jnp.dot(ha, w[:dh], preferred_element_type=_f32)
             + jnp.dot(hb, w[dh:], preferred_element_type=_f32))
        g_ref[...] = g
        es_ref[...] = lax.dot_general(at_ref[...], g, _DN_T,
                                      preferred_element_type=_f32)
        ed_ref[...] = lax.dot_general(ab_ref[...], g, _DN_T,
                                      preferred_element_type=_f32)

    return pl.pallas_call(
        body,
        grid=(ns // rblk,),
        in_specs=[
            pl.BlockSpec((2, rblk, dh), lambda i: (0, i, 0)),
            pl.BlockSpec((2 * dh, dout), lambda i: (0, 0)),
            pl.BlockSpec((dout, 1), lambda i: (0, 0)),
            pl.BlockSpec((dout, 1), lambda i: (0, 0)),
        ],
        out_specs=[
            pl.BlockSpec((rblk, dout), lambda i: (i, 0)),
            pl.BlockSpec((1, rblk), lambda i: (0, i)),
            pl.BlockSpec((1, rblk), lambda i: (0, i)),
        ],
        out_shape=[
            jax.ShapeDtypeStruct((ns, dout), _f32),
            jax.ShapeDtypeStruct((1, ns), _f32),
            jax.ShapeDtypeStruct((1, ns), _f32),
        ],
    )(p, W, at, ab)


def _final(p, Wl, bl, rblk=1280):
    ns = p.shape[1]
    dh = p.shape[2]
    dout = Wl.shape[1]

    def body(p_ref, w_ref, b_ref, o_ref):
        ha = p_ref[0]
        ha = jnp.where(ha > 0.0, ha, jnp.exp(ha) - 1.0)
        hb = p_ref[1]
        hb = jnp.where(hb > 0.0, hb, jnp.exp(hb) - 1.0)
        w = w_ref[...]
        logits = (jnp.dot(ha, w[:dh], preferred_element_type=_f32)
                  + jnp.dot(hb, w[dh:], preferred_element_type=_f32)
                  + b_ref[...])
        m = jnp.max(logits, axis=1, keepdims=True)
        lse = jnp.log(jnp.sum(jnp.exp(logits - m), axis=1, keepdims=True)) + m
        o_ref[...] = logits - lse

    return pl.pallas_call(
        body,
        grid=(ns // rblk,),
        in_specs=[
            pl.BlockSpec((2, rblk, dh), lambda i: (0, i, 0)),
            pl.BlockSpec((2 * dh, dout), lambda i: (0, 0)),
            pl.BlockSpec((1, dout), lambda i: (0, 0)),
        ],
        out_specs=pl.BlockSpec((rblk, dout), lambda i: (i, 0)),
        out_shape=jax.ShapeDtypeStruct((ns, dout), _f32),
    )(p, Wl, bl)


def kernel(input, edge, W0, a0, W1, a1, Wl, bl):
    x = jnp.pad(input.astype(_f32), ((0, NS - N), (0, 0)))
    pad = ES - E
    r = jnp.arange(pad, dtype=jnp.int32) % (NS - N)
    e3 = jnp.concatenate(
        [edge.astype(jnp.int32), jnp.stack([N + r, r])], axis=1
    ).reshape(2, ES // CH, CH)

    g0, es0, ed0 = _mm_proj(x, W0, a0[: 2 * HID], a0[2 * HID:])
    p0 = _edge_aggregate2(g0.reshape(2 * NS, HID), es0, ed0, e3,
                          jnp.zeros((NODES_PER_TILE, HID), _f32), HID)
    g1, es1, ed1 = _combine_mm(p0, W1, a1[:HID], a1[HID:])
    p1 = _edge_aggregate2(g1.reshape(2 * NS, HID // 2), es1, ed1, e3,
                          jnp.zeros((NODES_PER_TILE, HID // 2), _f32),
                          HID // 2)
    out = _final(p1, Wl, bl.reshape(1, OUT_DIM))
    return out[:N]


# speedup vs baseline: 29.5771x; 1.0163x over previous
import functools

import jax
import jax.numpy as jnp
from jax import lax
from jax.experimental import pallas as pl
from jax.experimental.pallas import tpu as pltpu
from jax.experimental.pallas import tpu_sc as plsc

N = 10000
E = 160000
IN_DIM = 128
HID = 64
OUT_DIM = 40
ALPHA = 0.2

NS = 10240
ES = 163840
CH = 128
NODES_PER_TILE = NS // 16

_f32 = jnp.float32
_DN_T = (((0,), (1,)), ((), ()))


def _sc_params():
    import dataclasses
    cp = pltpu.CompilerParams()
    fields = pltpu.CompilerParams.__dataclass_fields__
    if "needs_layout_passes" in fields:
        cp = dataclasses.replace(cp, needs_layout_passes=False)
    if "use_tc_tiling_on_sc" in fields:
        cp = dataclasses.replace(cp, use_tc_tiling_on_sc=False)
    if "disable_bounds_checks" in fields:
        cp = dataclasses.replace(cp, disable_bounds_checks=True)
    return cp


def _edge_aggregate2(g2, es, ed, e3, zrows, dh):
    mesh = plsc.VectorSubcoreMesh(core_axis_name="c", subcore_axis_name="s")
    R = (ES // CH) // 16
    HR = R // 2
    NBUF = 4
    NSB = 2

    @functools.partial(
        pl.kernel,
        out_type=jax.ShapeDtypeStruct((2, NS, dh), _f32),
        mesh=mesh,
        compiler_params=_sc_params(),
        scratch_types=[
            pltpu.VMEM((HR, CH), jnp.int32),
            pltpu.VMEM((HR, CH), jnp.int32),
            pltpu.VMEM((NBUF, CH), _f32),
            pltpu.VMEM((NS,), _f32),
            pltpu.VMEM((NS,), _f32),
            [pltpu.VMEM((CH, dh), _f32) for _ in range(NBUF)],
            [pltpu.VMEM((CH, dh), _f32) for _ in range(NSB)],
            pltpu.VMEM((NODES_PER_TILE,), _f32),
            pltpu.VMEM_SHARED((NS, dh), _f32),
            pltpu.VMEM_SHARED((NS,), _f32),
            [pltpu.SemaphoreType.DMA for _ in range(NBUF)],
            [pltpu.SemaphoreType.DMA for _ in range(NSB)],
            [pltpu.SemaphoreType.DMA for _ in range(NBUF)],
            pltpu.SemaphoreType.DMA,
        ],
    )
    def kern(g_h, es_h, ed_h, e3_h, z_h, p_h,
             src_v, dst_v, ex_v, tab_s, tab_d, gbuf, sbuf, z_v, p_sh, s_sh,
             gsem, ssem, xsem, lsem):
        cid = lax.axis_index("c")
        sid = lax.axis_index("s")
        rbase = sid * R
        nbase = sid * NODES_PER_TILE

        def load_and_prime(base):
            pltpu.sync_copy(e3_h.at[0, pl.ds(base, HR)], src_v)
            pltpu.sync_copy(e3_h.at[1, pl.ds(base, HR)], dst_v)

            @pl.loop(0, HR)
            def _(j):
                for c in range(CH // 16):
                    dst_v[j, pl.ds(c * 16, 16)] = (
                        dst_v[j, pl.ds(c * 16, 16)] * 2 + cid)

            for b in range(NBUF):
                pltpu.async_copy(g_h.at[dst_v.at[b]], gbuf[b], gsem[b])

        def drain(ex_full):
            for b in range(NSB):
                pltpu.make_async_copy(sbuf[b], p_sh.at[src_v.at[b]],
                                      ssem[b]).wait()
            for b in range(NBUF):
                if ex_full:
                    pltpu.make_async_copy(ex_v.at[b], s_sh.at[src_v.at[b]],
                                          xsem[b]).wait()

        ld = [
            pltpu.async_copy(es_h.at[0], tab_s, lsem),
            pltpu.async_copy(ed_h.at[0], tab_d, lsem),
        ]

        pltpu.sync_copy(z_h, p_sh.at[pl.ds(nbase, NODES_PER_TILE)])

        @pl.loop(0, NODES_PER_TILE, step=16)
        def _(i):
            z_v[pl.ds(i, 16)] = jnp.zeros((16,), _f32)

        pltpu.sync_copy(z_v, s_sh.at[pl.ds(nbase, NODES_PER_TILE)])

        for c in ld:
            c.wait()
        load_and_prime(rbase)

        plsc.subcore_barrier()

        for half in range(2):
            if half:
                drain(ex_full=True)
                load_and_prime(rbase + HR)

            @pl.loop(0, HR, step=NBUF)
            def _(j0):
                for b in range(NBUF):
                    j = j0 + b
                    @pl.when(j0 >= NBUF)
                    def _():
                        pltpu.make_async_copy(
                            ex_v.at[b], s_sh.at[src_v.at[j]],
                            xsem[b]).wait()

                    for c in range(CH // 16):
                        sidx = src_v[j, pl.ds(c * 16, 16)]
                        didx = lax.shift_right_logical(
                            dst_v[j, pl.ds(c * 16, 16)] - cid, 1)
                        e = (plsc.load_gather(tab_s, [sidx])
                             + plsc.load_gather(tab_d, [didx]))
                        v = jnp.where(e > 0.0, e, ALPHA * e)
                        ex_v[b, pl.ds(c * 16, 16)] = jnp.exp(v)
                    pltpu.async_copy(ex_v.at[b], s_sh.at[src_v.at[j]],
                                     xsem[b], add=True)

                    pltpu.make_async_copy(g_h.at[dst_v.at[j]], gbuf[b],
                                          gsem[b]).wait()
                    sb = b % NSB
                    if b >= NSB:
                        pltpu.make_async_copy(
                            sbuf[sb], p_sh.at[src_v.at[j]], ssem[sb]).wait()
                    else:
                        @pl.when(j0 >= NBUF)
                        def _():
                            pltpu.make_async_copy(
                                sbuf[sb], p_sh.at[src_v.at[j]],
                                ssem[sb]).wait()

                    @pl.loop(0, CH, step=16)
                    def _(rr):
                        a16 = ex_v[b, pl.ds(rr, 16)]
                        for t in range(16):
                            a = a16[t]
                            for q in range(dh // 16):
                                sbuf[sb][rr + t, pl.ds(q * 16, 16)] = (
                                    gbuf[b][rr + t, pl.ds(q * 16, 16)] * a)

                    pltpu.async_copy(sbuf[sb], p_sh.at[src_v.at[j]], ssem[sb],
                                     add=True)

                    @pl.when(j + NBUF < HR)
                    def _():
                        pltpu.async_copy(g_h.at[dst_v.at[j + NBUF]], gbuf[b],
                                         gsem[b])

        drain(ex_full=True)
        plsc.subcore_barrier()

        pltpu.sync_copy(s_sh.at[pl.ds(nbase, NODES_PER_TILE)], z_v)

        @pl.loop(0, NODES_PER_TILE, step=16)
        def _(i):
            s16 = z_v[pl.ds(i, 16)]
            z_v[pl.ds(i, 16)] = jnp.where(s16 > 0.0, 1.0 / s16, 0.0)

        @pl.loop(0, NODES_PER_TILE, step=CH)
        def _(i):
            pltpu.sync_copy(p_sh.at[pl.ds(nbase + i, CH)], gbuf[0])

            @pl.loop(0, CH, step=16)
            def _(rr):
                a16 = z_v[pl.ds(i + rr, 16)]
                for t in range(16):
                    a = a16[t]
                    for q in range(dh // 16):
                        sbuf[0][rr + t, pl.ds(q * 16, 16)] = (
                            gbuf[0][rr + t, pl.ds(q * 16, 16)] * a)

            pltpu.sync_copy(sbuf[0], p_h.at[cid, pl.ds(nbase + i, CH)])

    return kern(g2, es, ed, e3, zrows)


def _mm_proj(x, W, at, ab, rblk=1280):
    ns, k = x.shape
    dout = W.shape[1]

    def body(x_ref, w_ref, at_ref, ab_ref, g_ref, es_ref, ed_ref):
        g = jnp.dot(x_ref[...], w_ref[...], preferred_element_type=_f32)
        g_ref[...] = g
        es_ref[...] = lax.dot_general(at_ref[...], g, _DN_T,
                                      preferred_element_type=_f32)
        ed_ref[...] = lax.dot_general(ab_ref[...], g, _DN_T,
                                      preferred_element_type=_f32)

    return pl.pallas_call(
        body,
        grid=(ns // rblk,),
        in_specs=[
            pl.BlockSpec((rblk, k), lambda i: (i, 0)),
            pl.BlockSpec((k, dout), lambda i: (0, 0)),
            pl.BlockSpec((k, 1), lambda i: (0, 0)),
            pl.BlockSpec((k, 1), lambda i: (0, 0)),
        ],
        out_specs=[
            pl.BlockSpec((rblk, dout), lambda i: (i, 0)),
            pl.BlockSpec((1, rblk), lambda i: (0, i)),
            pl.BlockSpec((1, rblk), lambda i: (0, i)),
        ],
        out_shape=[
            jax.ShapeDtypeStruct((ns, dout), _f32),
            jax.ShapeDtypeStruct((1, ns), _f32),
            jax.ShapeDtypeStruct((1, ns), _f32),
        ],
    )(x, W, at, ab)


def _combine_mm(p, W, at, ab, rblk=1280):
    ns = p.shape[1]
    dh = p.shape[2]
    dout = W.shape[1]

    def body(p_ref, w_ref, at_ref, ab_ref, g_ref, es_ref, ed_ref):
        ha = p_ref[0]
        ha = jnp.where(ha > 0.0, ha, jnp.exp(ha) - 1.0)
        hb = p_ref[1]
        hb = jnp.where(hb > 0.0, hb, jnp.exp(hb) - 1.0)
        w = w_ref[...]
        g = (jnp.dot(ha, w[:dh], preferred_element_type=_f32)
             + jnp.dot(hb, w[dh:], preferred_element_type=_f32))
        g_ref[...] = g
        es_ref[...] = lax.dot_general(at_ref[...], g, _DN_T,
                                      preferred_element_type=_f32)
        ed_ref[...] = lax.dot_general(ab_ref[...], g, _DN_T,
                                      preferred_element_type=_f32)

    return pl.pallas_call(
        body,
        grid=(ns // rblk,),
        in_specs=[
            pl.BlockSpec((2, rblk, dh), lambda i: (0, i, 0)),
            pl.BlockSpec((2 * dh, dout), lambda i: (0, 0)),
            pl.BlockSpec((dout, 1), lambda i: (0, 0)),
            pl.BlockSpec((dout, 1), lambda i: (0, 0)),
        ],
        out_specs=[
            pl.BlockSpec((rblk, dout), lambda i: (i, 0)),
            pl.BlockSpec((1, rblk), lambda i: (0, i)),
            pl.BlockSpec((1, rblk), lambda i: (0, i)),
        ],
        out_shape=[
            jax.ShapeDtypeStruct((ns, dout), _f32),
            jax.ShapeDtypeStruct((1, ns), _f32),
            jax.ShapeDtypeStruct((1, ns), _f32),
        ],
    )(p, W, at, ab)


def _final(p, Wl, bl, rblk=1280):
    ns = p.shape[1]
    dh = p.shape[2]
    dout = Wl.shape[1]

    def body(p_ref, w_ref, b_ref, o_ref):
        ha = p_ref[0]
        ha = jnp.where(ha > 0.0, ha, jnp.exp(ha) - 1.0)
        hb = p_ref[1]
        hb = jnp.where(hb > 0.0, hb, jnp.exp(hb) - 1.0)
        w = w_ref[...]
        logits = (jnp.dot(ha, w[:dh], preferred_element_type=_f32)
                  + jnp.dot(hb, w[dh:], preferred_element_type=_f32)
                  + b_ref[...])
        m = jnp.max(logits, axis=1, keepdims=True)
        lse = jnp.log(jnp.sum(jnp.exp(logits - m), axis=1, keepdims=True)) + m
        o_ref[...] = logits - lse

    return pl.pallas_call(
        body,
        grid=(ns // rblk,),
        in_specs=[
            pl.BlockSpec((2, rblk, dh), lambda i: (0, i, 0)),
            pl.BlockSpec((2 * dh, dout), lambda i: (0, 0)),
            pl.BlockSpec((1, dout), lambda i: (0, 0)),
        ],
        out_specs=pl.BlockSpec((rblk, dout), lambda i: (i, 0)),
        out_shape=jax.ShapeDtypeStruct((ns, dout), _f32),
    )(p, Wl, bl)


def kernel(input, edge, W0, a0, W1, a1, Wl, bl):
    x = jnp.pad(input.astype(_f32), ((0, NS - N), (0, 0)))
    pad = ES - E
    r = jnp.arange(pad, dtype=jnp.int32) % (NS - N)
    e3 = jnp.concatenate(
        [edge.astype(jnp.int32), jnp.stack([N + r, r])], axis=1
    ).reshape(2, ES // CH, CH)

    g0, es0, ed0 = _mm_proj(x, W0, a0[: 2 * HID], a0[2 * HID:])
    p0 = _edge_aggregate2(g0.reshape(2 * NS, HID), es0, ed0, e3,
                          jnp.zeros((NODES_PER_TILE, HID), _f32), HID)
    g1, es1, ed1 = _combine_mm(p0, W1, a1[:HID], a1[HID:])
    p1 = _edge_aggregate2(g1.reshape(2 * NS, HID // 2), es1, ed1, e3,
                          jnp.zeros((NODES_PER_TILE, HID // 2), _f32),
                          HID // 2)
    out = _final(p1, Wl, bl.reshape(1, OUT_DIM))
    return out[:N]


# speedup vs baseline: 31.5153x; 1.0655x over previous
import functools

import jax
import jax.numpy as jnp
from jax import lax
from jax.experimental import pallas as pl
from jax.experimental.pallas import tpu as pltpu
from jax.experimental.pallas import tpu_sc as plsc

N = 10000
E = 160000
IN_DIM = 128
HID = 64
OUT_DIM = 40
ALPHA = 0.2

NS = 10240
ES = 163840
CH = 128
NODES_PER_TILE = NS // 16

_f32 = jnp.float32
_DN_T = (((0,), (1,)), ((), ()))


def _sc_params():
    import dataclasses
    cp = pltpu.CompilerParams()
    fields = pltpu.CompilerParams.__dataclass_fields__
    if "needs_layout_passes" in fields:
        cp = dataclasses.replace(cp, needs_layout_passes=False)
    if "use_tc_tiling_on_sc" in fields:
        cp = dataclasses.replace(cp, use_tc_tiling_on_sc=False)
    if "disable_bounds_checks" in fields:
        cp = dataclasses.replace(cp, disable_bounds_checks=True)
    return cp


def _edge_aggregate2(g2, esed, e3, zrows, dh, split_tabs=False):
    mesh = plsc.VectorSubcoreMesh(core_axis_name="c", subcore_axis_name="s")
    R = (ES // CH) // 16
    HR = R // 2
    NBUF = 4
    NSB = 2

    @functools.partial(
        pl.kernel,
        out_type=jax.ShapeDtypeStruct((2, NS, dh), _f32),
        mesh=mesh,
        compiler_params=_sc_params(),
        scratch_types=[
            pltpu.VMEM((HR, CH), jnp.int32),
            pltpu.VMEM((HR, CH), jnp.int32),
            pltpu.VMEM((NBUF, CH), _f32),
            [pltpu.VMEM((NS // 2,), _f32) for _ in range(4)]
            if split_tabs else
            [pltpu.VMEM((NS,), _f32) for _ in range(2)],
            [pltpu.VMEM((CH, dh), _f32) for _ in range(NBUF)],
            [pltpu.VMEM((CH, dh), _f32) for _ in range(NSB)],
            pltpu.VMEM((NODES_PER_TILE,), _f32),
            pltpu.VMEM_SHARED((NS, dh), _f32),
            pltpu.VMEM_SHARED((NS,), _f32),
            [pltpu.SemaphoreType.DMA for _ in range(NBUF)],
            [pltpu.SemaphoreType.DMA for _ in range(NSB)],
            [pltpu.SemaphoreType.DMA for _ in range(NBUF)],
            pltpu.SemaphoreType.DMA,
        ],
    )
    def kern(g_h, *refs):
        if split_tabs:
            (ese_h, eso_h, ede_h, edo_h, e3_h, z_h, p_h,
             src_v, dst_v, ex_v, tabs, gbuf, sbuf, z_v, p_sh, s_sh,
             gsem, ssem, xsem, lsem) = refs
        else:
            (es_h, ed_h, e3_h, z_h, p_h,
             src_v, dst_v, ex_v, tabs, gbuf, sbuf, z_v, p_sh, s_sh,
             gsem, ssem, xsem, lsem) = refs
        cid = lax.axis_index("c")
        sid = lax.axis_index("s")
        rbase = sid * R
        nbase = sid * NODES_PER_TILE

        def load_and_prime(base):
            pltpu.sync_copy(e3_h.at[0, pl.ds(base, HR)], src_v)
            pltpu.sync_copy(e3_h.at[1, pl.ds(base, HR)], dst_v)

            @pl.loop(0, HR)
            def _(j):
                for c in range(CH // 16):
                    dst_v[j, pl.ds(c * 16, 16)] = (
                        dst_v[j, pl.ds(c * 16, 16)] * 2 + cid)

            for b in range(NBUF):
                pltpu.async_copy(g_h.at[dst_v.at[b]], gbuf[b], gsem[b])

        def drain(ex_full):
            for b in range(NSB):
                pltpu.make_async_copy(sbuf[b], p_sh.at[src_v.at[b]],
                                      ssem[b]).wait()
            for b in range(NBUF):
                if ex_full:
                    pltpu.make_async_copy(ex_v.at[b], s_sh.at[src_v.at[b]],
                                          xsem[b]).wait()

        if split_tabs:
            srcs = (ese_h, eso_h, ede_h, edo_h)
        else:
            srcs = (es_h, ed_h)
        ld = [pltpu.async_copy(s.at[0], t, lsem) for s, t in zip(srcs, tabs)]

        pltpu.sync_copy(z_h, p_sh.at[pl.ds(nbase, NODES_PER_TILE)])

        @pl.loop(0, NODES_PER_TILE, step=16)
        def _(i):
            z_v[pl.ds(i, 16)] = jnp.zeros((16,), _f32)

        pltpu.sync_copy(z_v, s_sh.at[pl.ds(nbase, NODES_PER_TILE)])

        for c in ld:
            c.wait()
        load_and_prime(rbase)

        plsc.subcore_barrier()

        for half in range(2):
            if half:
                drain(ex_full=True)
                load_and_prime(rbase + HR)

            @pl.loop(0, HR, step=NBUF)
            def _(j0):
                for b in range(NBUF):
                    j = j0 + b
                    @pl.when(j0 >= NBUF)
                    def _():
                        pltpu.make_async_copy(
                            ex_v.at[b], s_sh.at[src_v.at[j]],
                            xsem[b]).wait()

                    for c in range(CH // 16):
                        sidx = src_v[j, pl.ds(c * 16, 16)]
                        didx = dst_v[j, pl.ds(c * 16, 16)]
                        if split_tabs:
                            sh = lax.shift_right_logical(sidx, 1)
                            sp = jnp.bitwise_and(sidx, 1)
                            esv = jnp.where(sp == 0,
                                            plsc.load_gather(tabs[0], [sh]),
                                            plsc.load_gather(tabs[1], [sh]))
                            dh_i = lax.shift_right_logical(didx, 2)
                            dp = jnp.bitwise_and(
                                lax.shift_right_logical(didx, 1), 1)
                            edv = jnp.where(dp == 0,
                                            plsc.load_gather(tabs[2], [dh_i]),
                                            plsc.load_gather(tabs[3], [dh_i]))
                            e = esv + edv
                        else:
                            dn = lax.shift_right_logical(didx - cid, 1)
                            e = (plsc.load_gather(tabs[0], [sidx])
                                 + plsc.load_gather(tabs[1], [dn]))
                        v = jnp.where(e > 0.0, e, ALPHA * e)
                        ex_v[b, pl.ds(c * 16, 16)] = jnp.exp(v)
                    pltpu.async_copy(ex_v.at[b], s_sh.at[src_v.at[j]],
                                     xsem[b], add=True)

                    pltpu.make_async_copy(g_h.at[dst_v.at[j]], gbuf[b],
                                          gsem[b]).wait()
                    sb = b % NSB
                    if b >= NSB:
                        pltpu.make_async_copy(
                            sbuf[sb], p_sh.at[src_v.at[j]], ssem[sb]).wait()
                    else:
                        @pl.when(j0 >= NBUF)
                        def _():
                            pltpu.make_async_copy(
                                sbuf[sb], p_sh.at[src_v.at[j]],
                                ssem[sb]).wait()

                    @pl.loop(0, CH, step=16)
                    def _(rr):
                        a16 = ex_v[b, pl.ds(rr, 16)]
                        for t in range(16):
                            a = a16[t]
                            for q in range(dh // 16):
                                sbuf[sb][rr + t, pl.ds(q * 16, 16)] = (
                                    gbuf[b][rr + t, pl.ds(q * 16, 16)] * a)

                    pltpu.async_copy(sbuf[sb], p_sh.at[src_v.at[j]], ssem[sb],
                                     add=True)

                    @pl.when(j + NBUF < HR)
                    def _():
                        pltpu.async_copy(g_h.at[dst_v.at[j + NBUF]], gbuf[b],
                                         gsem[b])

        drain(ex_full=True)
        plsc.subcore_barrier()

        pltpu.sync_copy(s_sh.at[pl.ds(nbase, NODES_PER_TILE)], z_v)

        @pl.loop(0, NODES_PER_TILE, step=16)
        def _(i):
            s16 = z_v[pl.ds(i, 16)]
            z_v[pl.ds(i, 16)] = jnp.where(s16 > 0.0, 1.0 / s16, 0.0)

        @pl.loop(0, NODES_PER_TILE, step=CH)
        def _(i):
            pltpu.sync_copy(p_sh.at[pl.ds(nbase + i, CH)], gbuf[0])

            @pl.loop(0, CH, step=16)
            def _(rr):
                a16 = z_v[pl.ds(i + rr, 16)]
                for t in range(16):
                    a = a16[t]
                    for q in range(dh // 16):
                        sbuf[0][rr + t, pl.ds(q * 16, 16)] = (
                            gbuf[0][rr + t, pl.ds(q * 16, 16)] * a)

            pltpu.sync_copy(sbuf[0], p_h.at[cid, pl.ds(nbase + i, CH)])

    return kern(g2, *esed, e3, zrows)


def _mm_proj(x, W, at, ab, rblk=1280):
    ns, k = x.shape
    dout = W.shape[1]

    def body(x_ref, w_ref, at_ref, ab_ref, g_ref, es_ref, ed_ref):
        g = jnp.dot(x_ref[...], w_ref[...], preferred_element_type=_f32)
        g_ref[...] = g
        es_ref[...] = lax.dot_general(at_ref[...], g, _DN_T,
                                      preferred_element_type=_f32)
        ed_ref[...] = lax.dot_general(ab_ref[...], g, _DN_T,
                                      preferred_element_type=_f32)

    return pl.pallas_call(
        body,
        grid=(ns // rblk,),
        in_specs=[
            pl.BlockSpec((rblk, k), lambda i: (i, 0)),
            pl.BlockSpec((k, dout), lambda i: (0, 0)),
            pl.BlockSpec((k, 1), lambda i: (0, 0)),
            pl.BlockSpec((k, 1), lambda i: (0, 0)),
        ],
        out_specs=[
            pl.BlockSpec((rblk, dout), lambda i: (i, 0)),
            pl.BlockSpec((1, rblk), lambda i: (0, i)),
            pl.BlockSpec((1, rblk), lambda i: (0, i)),
        ],
        out_shape=[
            jax.ShapeDtypeStruct((ns, dout), _f32),
            jax.ShapeDtypeStruct((1, ns), _f32),
            jax.ShapeDtypeStruct((1, ns), _f32),
        ],
    )(x, W, at, ab)


def _bd(M):
    z = jnp.zeros_like(M)
    return jnp.concatenate(
        [jnp.concatenate([M, z], axis=1), jnp.concatenate([z, M], axis=1)],
        axis=0)


def _combine_mm(pp, W, at, ab, rblk=1280):
    ns2 = pp.shape[1]
    dh = 64
    dout = W.shape[1]
    r2 = rblk // 2
    Wt = _bd(W[:dh])
    Wb = _bd(W[dh:])
    z = jnp.zeros((dout, 1), _f32)
    at_e = jnp.concatenate([at, z], axis=0)
    at_o = jnp.concatenate([z, at], axis=0)
    ab_e = jnp.concatenate([ab, z], axis=0)
    ab_o = jnp.concatenate([z, ab], axis=0)

    def body(pp_ref, wt_ref, wb_ref, ate_ref, ato_ref, abe_ref, abo_ref,
             g_ref, ese_ref, eso_ref, ede_ref, edo_ref):
        ha = pp_ref[0]
        ha = jnp.where(ha > 0.0, ha, jnp.exp(ha) - 1.0)
        hb = pp_ref[1]
        hb = jnp.where(hb > 0.0, hb, jnp.exp(hb) - 1.0)
        g = (jnp.dot(ha, wt_ref[...], preferred_element_type=_f32)
             + jnp.dot(hb, wb_ref[...], preferred_element_type=_f32))
        g_ref[...] = g
        ese_ref[...] = lax.dot_general(ate_ref[...], g, _DN_T,
                                       preferred_element_type=_f32)
        eso_ref[...] = lax.dot_general(ato_ref[...], g, _DN_T,
                                       preferred_element_type=_f32)
        ede_ref[...] = lax.dot_general(abe_ref[...], g, _DN_T,
                                       preferred_element_type=_f32)
        edo_ref[...] = lax.dot_general(abo_ref[...], g, _DN_T,
                                       preferred_element_type=_f32)

    full = lambda i: (0, 0)
    row = pl.BlockSpec((1, r2), lambda i: (0, i))
    return pl.pallas_call(
        body,
        grid=(ns2 // r2,),
        in_specs=[
            pl.BlockSpec((2, r2, 128), lambda i: (0, i, 0)),
            pl.BlockSpec((128, 128), full),
            pl.BlockSpec((128, 128), full),
            pl.BlockSpec((128, 1), full),
            pl.BlockSpec((128, 1), full),
            pl.BlockSpec((128, 1), full),
            pl.BlockSpec((128, 1), full),
        ],
        out_specs=[pl.BlockSpec((r2, 128), lambda i: (i, 0)),
                   row, row, row, row],
        out_shape=[
            jax.ShapeDtypeStruct((ns2, 128), _f32),
            jax.ShapeDtypeStruct((1, ns2), _f32),
            jax.ShapeDtypeStruct((1, ns2), _f32),
            jax.ShapeDtypeStruct((1, ns2), _f32),
            jax.ShapeDtypeStruct((1, ns2), _f32),
        ],
    )(pp, Wt, Wb, at_e, at_o, ab_e, ab_o)


def _final(p, Wl, bl, rblk=1280):
    ns = p.shape[1]
    dh = p.shape[2]
    dout = Wl.shape[1]

    def body(p_ref, w_ref, b_ref, o_ref):
        ha = p_ref[0]
        ha = jnp.where(ha > 0.0, ha, jnp.exp(ha) - 1.0)
        hb = p_ref[1]
        hb = jnp.where(hb > 0.0, hb, jnp.exp(hb) - 1.0)
        w = w_ref[...]
        logits = (jnp.dot(ha, w[:dh], preferred_element_type=_f32)
                  + jnp.dot(hb, w[dh:], preferred_element_type=_f32)
                  + b_ref[...])
        m = jnp.max(logits, axis=1, keepdims=True)
        lse = jnp.log(jnp.sum(jnp.exp(logits - m), axis=1, keepdims=True)) + m
        o_ref[...] = logits - lse

    return pl.pallas_call(
        body,
        grid=(ns // rblk,),
        in_specs=[
            pl.BlockSpec((2, rblk, dh), lambda i: (0, i, 0)),
            pl.BlockSpec((2 * dh, dout), lambda i: (0, 0)),
            pl.BlockSpec((1, dout), lambda i: (0, 0)),
        ],
        out_specs=pl.BlockSpec((rblk, dout), lambda i: (i, 0)),
        out_shape=jax.ShapeDtypeStruct((ns, dout), _f32),
    )(p, Wl, bl)


def kernel(input, edge, W0, a0, W1, a1, Wl, bl):
    x = jnp.pad(input.astype(_f32), ((0, NS - N), (0, 0)))
    pad = ES - E
    r = jnp.arange(pad, dtype=jnp.int32) % (NS - N)
    e3 = jnp.concatenate(
        [edge.astype(jnp.int32), jnp.stack([N + r, r])], axis=1
    ).reshape(2, ES // CH, CH)

    g0, es0, ed0 = _mm_proj(x, W0, a0[: 2 * HID], a0[2 * HID:])
    p0 = _edge_aggregate2(g0.reshape(2 * NS, HID), (es0, ed0), e3,
                          jnp.zeros((NODES_PER_TILE, HID), _f32), HID)
    g1, ese, eso, ede, edo = _combine_mm(p0.reshape(2, NS // 2, 128),
                                         W1, a1[:HID], a1[HID:])
    p1 = _edge_aggregate2(g1.reshape(2 * NS, HID // 2), (ese, eso, ede, edo),
                          e3, jnp.zeros((NODES_PER_TILE, HID // 2), _f32),
                          HID // 2, split_tabs=True)
    out = _final(p1, Wl, bl.reshape(1, OUT_DIM))
    return out[:N]
